# baseline (speedup 1.0000x reference)
import functools

import jax
import jax.numpy as jnp
from jax import lax
from jax.experimental import pallas as pl
from jax.experimental.pallas import tpu as pltpu
from jax.experimental.pallas import tpu_sc as plsc

HIDDEN = 2048
LANES = 16
NVEC = HIDDEN // LANES
ROWS = 16384
CHUNK = 8


def _sc_body(x_hbm, perm_hbm, signs_hbm, mix_hbm, out_hbm,
             perm_v, signs_v, mix_v, in_v, out_v):
    nc = 2
    wid = lax.axis_index("s") * nc + lax.axis_index("c")
    nworkers = 32
    rows_per_w = ROWS // nworkers
    base_row = wid * rows_per_w

    pltpu.sync_copy(perm_hbm, perm_v)
    pltpu.sync_copy(signs_hbm, signs_v)
    pltpu.sync_copy(mix_hbm, mix_v)

    mv = mix_v[...]
    av = 1.0 - mv

    def _premul(j, _):
        sl = pl.ds(j * LANES, LANES)
        signs_v[sl] = signs_v[sl] * mv
        return _

    lax.fori_loop(0, NVEC, _premul, None)

    nchunks = rows_per_w // CHUNK

    def _chunk(c, _):
        elem0 = (base_row + c * CHUNK) * HIDDEN
        pltpu.sync_copy(x_hbm.at[pl.ds(elem0, CHUNK * HIDDEN)], in_v)

        def _col(j, _):
            sl = pl.ds(j * LANES, LANES)
            idxv = perm_v[sl]
            msv = signs_v[sl]
            for r in range(CHUNK):
                g = plsc.load_gather(in_v, [idxv + (r * HIDDEN)])
                xv = in_v[pl.ds(r * HIDDEN + j * LANES, LANES)]
                out_v[pl.ds(r * HIDDEN + j * LANES, LANES)] = av * xv + msv * g
            return _

        lax.fori_loop(0, NVEC, _col, None)
        pltpu.sync_copy(out_v, out_hbm.at[pl.ds(elem0, CHUNK * HIDDEN)])
        return _

    lax.fori_loop(0, nchunks, _chunk, None)


@functools.partial(jax.jit, static_argnames=())
def kernel(x, perm, signs, mix):
    orig_shape = x.shape
    x1d = x.reshape(ROWS * HIDDEN)
    perm32 = perm.astype(jnp.int32)
    mix_vec = jnp.broadcast_to(mix.astype(jnp.float32), (LANES,))

    mesh = plsc.VectorSubcoreMesh(core_axis_name="c", subcore_axis_name="s")
    out1d = pl.kernel(
        _sc_body,
        out_type=jax.ShapeDtypeStruct((ROWS * HIDDEN,), jnp.float32),
        mesh=mesh,
        scratch_types=[
            pltpu.VMEM((HIDDEN,), jnp.int32),
            pltpu.VMEM((HIDDEN,), jnp.float32),
            pltpu.VMEM((LANES,), jnp.float32),
            pltpu.VMEM((CHUNK * HIDDEN,), jnp.float32),
            pltpu.VMEM((CHUNK * HIDDEN,), jnp.float32),
        ],
        compiler_params=pltpu.CompilerParams(needs_layout_passes=False),
    )(x1d, perm32, signs.astype(jnp.float32), mix_vec)
    return out1d.reshape(orig_shape)


# speedup vs baseline: 1.1811x; 1.1811x over previous
import functools

import jax
import jax.numpy as jnp
from jax import lax
from jax.experimental import pallas as pl
from jax.experimental.pallas import tpu as pltpu
from jax.experimental.pallas import tpu_sc as plsc

HIDDEN = 2048
LANES = 16
NVEC = HIDDEN // LANES
ROWS = 16384
CHUNK = 8
NBUF = 2


def _sc_body(x_hbm, perm_hbm, signs_hbm, mix_hbm, out_hbm,
             perm_v, signs_v, mix_v,
             in_v0, in_v1, out_v0, out_v1,
             in_sem0, in_sem1, out_sem0, out_sem1):
    in_v = (in_v0, in_v1)
    out_v = (out_v0, out_v1)
    in_sem = (in_sem0, in_sem1)
    out_sem = (out_sem0, out_sem1)
    nc = 2
    wid = lax.axis_index("s") * nc + lax.axis_index("c")
    nworkers = 32
    rows_per_w = ROWS // nworkers
    base_row = wid * rows_per_w

    pltpu.sync_copy(perm_hbm, perm_v)
    pltpu.sync_copy(signs_hbm, signs_v)
    pltpu.sync_copy(mix_hbm, mix_v)

    mv = mix_v[...]
    av = 1.0 - mv

    def _premul(j, _):
        sl = pl.ds(j * LANES, LANES)
        signs_v[sl] = signs_v[sl] * mv
        return _

    lax.fori_loop(0, NVEC, _premul, None)

    nchunks = rows_per_w // CHUNK
    celems = CHUNK * HIDDEN

    def _in_copy(chunk_idx, b):
        elem0 = (base_row * HIDDEN) + chunk_idx * celems
        return pltpu.make_async_copy(
            x_hbm.at[pl.ds(elem0, celems)], in_v[b], in_sem[b])

    def _out_copy(chunk_idx, b):
        elem0 = (base_row * HIDDEN) + chunk_idx * celems
        return pltpu.make_async_copy(
            out_v[b], out_hbm.at[pl.ds(elem0, celems)], out_sem[b])

    _in_copy(0, 0).start()
    _in_copy(1, 1).start()

    def _pair(p, _):
        for b in range(NBUF):
            g = p * NBUF + b
            _in_copy(g, b).wait()

            @pl.when(p > 0)
            def _():
                _out_copy(g - NBUF, b).wait()

            def _col(j, _):
                sl = pl.ds(j * LANES, LANES)
                idxv = perm_v[sl]
                msv = signs_v[sl]
                for r in range(CHUNK):
                    gat = plsc.load_gather(in_v[b], [idxv + (r * HIDDEN)])
                    xv = in_v[b][pl.ds(r * HIDDEN + j * LANES, LANES)]
                    out_v[b][pl.ds(r * HIDDEN + j * LANES, LANES)] = (
                        av * xv + msv * gat)
                return _

            lax.fori_loop(0, NVEC, _col, None)
            _out_copy(g, b).start()

            @pl.when(g + NBUF < nchunks)
            def _():
                _in_copy(g + NBUF, b).start()
        return _

    lax.fori_loop(0, nchunks // NBUF, _pair, None)
    for b in range(NBUF):
        _out_copy(nchunks - NBUF + b, b).wait()


@functools.partial(jax.jit, static_argnames=())
def kernel(x, perm, signs, mix):
    orig_shape = x.shape
    x1d = x.reshape(ROWS * HIDDEN)
    perm32 = perm.astype(jnp.int32)
    mix_vec = jnp.broadcast_to(mix.astype(jnp.float32), (LANES,))

    mesh = plsc.VectorSubcoreMesh(core_axis_name="c", subcore_axis_name="s")
    out1d = pl.kernel(
        _sc_body,
        out_type=jax.ShapeDtypeStruct((ROWS * HIDDEN,), jnp.float32),
        mesh=mesh,
        scratch_types=[
            pltpu.VMEM((HIDDEN,), jnp.int32),
            pltpu.VMEM((HIDDEN,), jnp.float32),
            pltpu.VMEM((LANES,), jnp.float32),
            pltpu.VMEM((CHUNK * HIDDEN,), jnp.float32),
            pltpu.VMEM((CHUNK * HIDDEN,), jnp.float32),
            pltpu.VMEM((CHUNK * HIDDEN,), jnp.float32),
            pltpu.VMEM((CHUNK * HIDDEN,), jnp.float32),
            pltpu.SemaphoreType.DMA,
            pltpu.SemaphoreType.DMA,
            pltpu.SemaphoreType.DMA,
            pltpu.SemaphoreType.DMA,
        ],
        compiler_params=pltpu.CompilerParams(needs_layout_passes=False),
    )(x1d, perm32, signs.astype(jnp.float32), mix_vec)
    return out1d.reshape(orig_shape)
